# Initial kernel scaffold; baseline (speedup 1.0000x reference)
import jax
import jax.numpy as jnp
from jax.experimental import pallas as pl


def kernel(features, edge_index, f_idx, f_val, alpha_proj, alpha_bil, alpha_W, update_scale):
    raise NotImplementedError("write your pallas kernel here")


# speedup vs baseline: 35.6886x; 35.6886x over previous
import functools

import jax
import jax.numpy as jnp
from jax import lax
from jax.experimental import pallas as pl
from jax.experimental.pallas import tpu as pltpu
from jax.experimental.pallas import tpu_sc as plsc

N = 10000
E = 160000
D = 248
DP = 256
H = DP // 2
NNZ_PAD = 3840

NC, NS = 2, 16
CHUNK_E = 128
NCH = 79
EP = NS * NCH * CHUNK_E
NA = 10240
STRIPE = NA // NS


def _segment_sum_sc(feat2, src2, tgt3):
    mesh = plsc.VectorSubcoreMesh(core_axis_name="c", subcore_axis_name="s")

    @functools.partial(
        pl.kernel,
        out_type=jax.ShapeDtypeStruct((NC * NA, H), jnp.float32),
        mesh=mesh,
        scratch_types=[
            pltpu.VMEM_SHARED((NA, H), jnp.float32),
            pltpu.VMEM((NCH, CHUNK_E), jnp.int32),
            pltpu.VMEM((NCH, CHUNK_E), jnp.int32),
            pltpu.VMEM((CHUNK_E, H), jnp.float32),
            pltpu.SemaphoreType.DMA,
        ],
    )
    def seg(feat2_hbm, src2_hbm, tgt3_hbm, out_hbm, acc_sh, src_v, tgt_v,
            rows_v, sem):
        c = lax.axis_index("c")
        s = lax.axis_index("s")
        w = c * NS + s

        def zrow(r, carry):
            for q in range(H // 16):
                rows_v[r, pl.ds(q * 16, 16)] = jnp.zeros((16,), jnp.float32)
            return carry
        lax.fori_loop(0, CHUNK_E, zrow, 0)
        for t in range(STRIPE // CHUNK_E):
            pltpu.sync_copy(
                rows_v, acc_sh.at[pl.ds(s * STRIPE + t * CHUNK_E, CHUNK_E)])

        pltpu.sync_copy(src2_hbm.at[w], src_v)
        pltpu.sync_copy(tgt3_hbm.at[s], tgt_v)

        plsc.subcore_barrier()

        def step(j, carry):
            pltpu.async_copy(feat2_hbm.at[src_v.at[j]], rows_v, sem).wait()
            pltpu.sync_copy(rows_v, acc_sh.at[tgt_v.at[j]], add=True)
            return carry
        lax.fori_loop(0, NCH, step, 0)

        plsc.subcore_barrier()

        base = c * NA + s * STRIPE
        for t in range(STRIPE // CHUNK_E):
            pltpu.sync_copy(
                acc_sh.at[pl.ds(s * STRIPE + t * CHUNK_E, CHUNK_E)], rows_v)
            pltpu.sync_copy(
                rows_v, out_hbm.at[pl.ds(base + t * CHUNK_E, CHUNK_E)])

    return seg(feat2, src2, tgt3)


def _bracket_update_tc(xp, s_mat, gi, gj, fvs, hk):
    BR = 1000

    def body(x_ref, s_ref, gi_ref, gj_ref, fv_ref, hk_ref, o_ref):
        a = jnp.dot(s_ref[...], gi_ref[...],
                    preferred_element_type=jnp.float32)
        b = jnp.dot(x_ref[...], gj_ref[...],
                    preferred_element_type=jnp.float32)
        t = a * b * fv_ref[...]
        o_ref[...] = x_ref[...] + jnp.dot(
            t, hk_ref[...], preferred_element_type=jnp.float32)

    return pl.pallas_call(
        body,
        grid=(N // BR,),
        in_specs=[
            pl.BlockSpec((BR, DP), lambda i: (i, 0)),
            pl.BlockSpec((BR, DP), lambda i: (i, 0)),
            pl.BlockSpec((DP, NNZ_PAD), lambda i: (0, 0)),
            pl.BlockSpec((DP, NNZ_PAD), lambda i: (0, 0)),
            pl.BlockSpec((1, NNZ_PAD), lambda i: (0, 0)),
            pl.BlockSpec((NNZ_PAD, DP), lambda i: (0, 0)),
        ],
        out_specs=pl.BlockSpec((BR, DP), lambda i: (i, 0)),
        out_shape=jax.ShapeDtypeStruct((N, DP), jnp.float32),
    )(xp, s_mat, gi, gj, fvs, hk)


def kernel(features, edge_index, f_idx, f_val, alpha_proj, alpha_bil,
           alpha_W, update_scale):
    del alpha_W, update_scale

    xp = jnp.pad(features, ((0, 0), (0, DP - D)))
    feat2 = jnp.concatenate([xp[:, :H], xp[:, H:]], axis=0)

    src = edge_index[0]
    tgt = edge_index[1]
    src_p = jnp.pad(src, (0, EP - E))
    tgt_p = jnp.pad(tgt, (0, EP - E), constant_values=N)
    src3 = src_p.reshape(NS, NCH, CHUNK_E)
    src2 = jnp.concatenate([src3, src3 + N], axis=0)
    tgt3 = tgt_p.reshape(NS, NCH, CHUNK_E)

    seg = _segment_sum_sc(feat2, src2, tgt3)
    s_mat = jnp.concatenate([seg[:N], seg[NA:NA + N]], axis=1)

    nnz = f_idx.shape[0]
    fi = jnp.pad(f_idx[:, 0], (0, NNZ_PAD - nnz))
    fj = jnp.pad(f_idx[:, 1], (0, NNZ_PAD - nnz))
    fk = jnp.pad(f_idx[:, 2], (0, NNZ_PAD - nnz))
    fv = jnp.pad(f_val, (0, NNZ_PAD - nnz))
    ar = jnp.arange(DP, dtype=f_idx.dtype)
    gi = (fi[None, :] == ar[:, None]).astype(jnp.float32)
    gj = (fj[None, :] == ar[:, None]).astype(jnp.float32)
    hk = (fk[:, None] == ar[None, :]).astype(jnp.float32)
    fvs = (alpha_bil * alpha_proj * fv)[None, :]

    upd = _bracket_update_tc(xp, s_mat, gi, gj, fvs, hk)
    return upd[:, :D]
